# Initial kernel scaffold; baseline (speedup 1.0000x reference)
import jax
import jax.numpy as jnp
from jax.experimental import pallas as pl


def kernel(x, x_len, embed_weight):
    raise NotImplementedError("write your pallas kernel here")


# speedup vs baseline: 2.7240x; 2.7240x over previous
import functools

import jax
import jax.numpy as jnp
from jax import lax
from jax.experimental import pallas as pl
from jax.experimental.pallas import tpu as pltpu
from jax.experimental.pallas import tpu_sc as plsc

_B = 16384
_L = 50
_D = 32
_NW = 32
_BPW = _B // _NW
_R = 32
_NCHUNK = _BPW // _R
_NIDX = _R * _L


def _body(x_hbm, xlen_hbm, tab_hbm, out_hbm,
          xlen_v, idx_v, rows_v, out_v, sem):
    wid = lax.axis_index("s") * 2 + lax.axis_index("c")
    b0 = wid * _BPW

    pltpu.sync_copy(xlen_hbm.at[pl.ds(b0, _BPW)], xlen_v.at[pl.ds(0, _BPW)])

    z16 = jnp.zeros((16,), jnp.float32)

    def chunk_body(c, carry):
        cb = c * _R

        pltpu.sync_copy(x_hbm.at[pl.ds((b0 + cb) * _L, _NIDX)],
                        idx_v.at[pl.ds(0, _NIDX)])

        descs = []
        for r in range(12):
            descs.append(pltpu.async_copy(
                tab_hbm.at[idx_v.at[pl.ds(r * 128, 128)]],
                rows_v.at[pl.ds(r * 128, 128)], sem))
        descs.append(pltpu.async_copy(
            tab_hbm.at[idx_v.at[pl.ds(1536, 64)]],
            rows_v.at[pl.ds(1536, 64)], sem))
        for dsc in descs:
            dsc.wait()

        def row_body(i, inner_carry):
            base = i * _L
            ln = xlen_v[pl.ds(cb + i, 16)][0]

            def jbody(j, acc):
                a0, a1 = acc
                v = idx_v[pl.ds(base + j, 16)][0]
                r0 = rows_v[base + j, pl.ds(0, 16)]
                r1 = rows_v[base + j, pl.ds(16, 16)]
                nz = v != 0
                a0 = a0 + jnp.where(nz, r0, z16)
                a1 = a1 + jnp.where(nz, r1, z16)
                return (a0, a1)

            a0, a1 = lax.fori_loop(0, ln, jbody, (z16, z16))
            lnv = jnp.broadcast_to(ln, (16,))
            lnf = lnv.astype(jnp.float32)
            rv = jnp.where(lnv > 0, 1.0 / lnf, jnp.zeros((16,), jnp.float32))
            out_v[pl.ds(i * _D, 16)] = a0 * rv
            out_v[pl.ds(i * _D + 16, 16)] = a1 * rv
            return inner_carry

        lax.fori_loop(0, _R, row_body, 0)

        pltpu.sync_copy(out_v, out_hbm.at[pl.ds((b0 + cb) * _D, _R * _D)])
        return carry

    lax.fori_loop(0, _NCHUNK, chunk_body, 0)


@jax.jit
def kernel(x, x_len, embed_weight):
    x_flat = x.astype(jnp.int32).reshape(-1)
    mesh = plsc.VectorSubcoreMesh(core_axis_name="c", subcore_axis_name="s")
    f = functools.partial(
        pl.kernel,
        out_type=jax.ShapeDtypeStruct((_B * _D,), jnp.float32),
        mesh=mesh,
        compiler_params=pltpu.CompilerParams(use_tc_tiling_on_sc=False),
        scratch_types=[
            pltpu.VMEM((_BPW + 16,), jnp.int32),
            pltpu.VMEM((_NIDX + 16,), jnp.int32),
            pltpu.VMEM((_NIDX, _D), jnp.float32),
            pltpu.VMEM((_R * _D,), jnp.float32),
            pltpu.SemaphoreType.DMA,
        ],
    )(_body)
    out = f(x_flat, x_len.astype(jnp.int32), embed_weight)
    return out.reshape(_B, _D)
